# baseline (speedup 1.0000x reference)
import jax
import jax.numpy as jnp
from jax import lax
from jax.experimental import pallas as pl
from jax.experimental.pallas import tpu as pltpu

B, T_ENC, T_DEC = 64, 192, 600
VOCAB, N_MELS = 26, 80
EMB = 512; ENC = 512; K_ENC = 5; N_ENC_CONV = 3
PRE = 256; DEC = 512; ATT = 128; LOC_F = 32; LOC_K = 31
POST_C = 512; K_POST = 5; N_POST = 5


def _conv1d(x, w, b=None, pad=0):
    y = lax.conv_general_dilated(x, w, (1,), [(pad, pad)],
                                 dimension_numbers=('NCH', 'OIH', 'NCH'))
    return y if b is None else y + b[None, :, None]


def _lstm(xs, W):
    H = W['Wh'].shape[1]

    def step(carry, x):
        h, c = carry
        z = x @ W['Wi'].T + h @ W['Wh'].T + W['b']
        i, f, g, o = jnp.split(z, 4, -1)
        c = jax.nn.sigmoid(f) * c + jax.nn.sigmoid(i) * jnp.tanh(g)
        h = jax.nn.sigmoid(o) * jnp.tanh(c)
        return (h, c), h

    Bsz = xs.shape[1]
    init = (jnp.zeros((Bsz, H), xs.dtype), jnp.zeros((Bsz, H), xs.dtype))
    return lax.scan(step, init, xs)[1]


def _residual_add_kernel(a_ref, b_ref, o_ref):
    o_ref[...] = a_ref[...] + b_ref[...]


def _residual_add(a, b):
    return pl.pallas_call(
        _residual_add_kernel,
        out_shape=jax.ShapeDtypeStruct(a.shape, a.dtype),
        name="residual_add",
    )(a, b)


def kernel(texts, encoder_mask, mels_target, params):
    p = params
    x = p['emb'][texts].transpose(0, 2, 1)
    for c in p['enc_convs']:
        x = _conv1d(x, c['w'], c['b'], pad=K_ENC // 2)
        x = jax.nn.relu(x * c['s'][None, :, None] + c['o'][None, :, None])
    xs = x.transpose(2, 0, 1)
    hf = _lstm(xs, p['lstm_f'])
    hb = _lstm(xs[::-1], p['lstm_b'])[::-1]
    memory = jnp.concatenate([hf, hb], -1).transpose(1, 0, 2)
    pm = jnp.einsum('btd,ad->bta', memory, p['att_mem'])

    dec_in = jnp.concatenate(
        [jnp.zeros((B, 1, N_MELS), mels_target.dtype), mels_target[:, :-1]], 1)
    xs_dec = dec_in.transpose(1, 0, 2)
    W = p['cell']
    neg = jnp.float32(-1e9)

    def step(carry, frame):
        h, c, ctx, aw, acum = carry
        pre = jax.nn.relu(jax.nn.relu(frame @ p['pre1'].T) @ p['pre2'].T)
        z = jnp.concatenate([pre, ctx], -1) @ W['Wi'].T + h @ W['Wh'].T + W['b']
        i, f, g, o = jnp.split(z, 4, -1)
        c = jax.nn.sigmoid(f) * c + jax.nn.sigmoid(i) * jnp.tanh(g)
        h = jax.nn.sigmoid(o) * jnp.tanh(c)
        cat = jnp.stack([aw, acum], 1)
        loc = _conv1d(cat, p['att_loc_conv'], pad=LOC_K // 2)
        loc = jnp.einsum('bft,af->bta', loc, p['att_loc_lin'])
        q = (h @ p['att_q'].T)[:, None, :]
        e = jnp.tanh(q + pm + loc) @ p['att_v']
        e = jnp.where(encoder_mask, e, neg)
        al = jax.nn.softmax(e, -1)
        ctx = jnp.einsum('bt,btd->bd', al, memory)
        dc = jnp.concatenate([h, ctx], -1)
        fr = dc @ p['proj_w'].T + p['proj_b']
        gt = (dc @ p['gate_w'].T + p['gate_b'])[:, 0]
        return (h, c, ctx, al, acum + al), (fr, gt, al)

    init = (jnp.zeros((B, DEC)), jnp.zeros((B, DEC)), jnp.zeros((B, ENC)),
            jnp.zeros((B, T_ENC)), jnp.zeros((B, T_ENC)))
    _, (frames, gates, aligns) = lax.scan(step, init, xs_dec)
    dec_out = frames.transpose(1, 0, 2)
    gate_out = gates.transpose(1, 0)
    alignments = aligns.transpose(1, 0, 2)

    y = dec_out.transpose(0, 2, 1)
    for idx, c in enumerate(p['postnet']):
        y = _conv1d(y, c['w'], c['b'], pad=K_POST // 2)
        y = y * c['s'][None, :, None] + c['o'][None, :, None]
        if idx < N_POST - 1:
            y = jnp.tanh(y)
    post_out = _residual_add(y.transpose(0, 2, 1), dec_out)
    return dec_out, post_out, alignments, gate_out


# speedup vs baseline: 3.1166x; 3.1166x over previous
import functools

import jax
import jax.numpy as jnp
from jax import lax
from jax.experimental import pallas as pl
from jax.experimental.pallas import tpu as pltpu

B, T_ENC, T_DEC = 64, 192, 600
VOCAB, N_MELS = 26, 80
EMB = 512; ENC = 512; K_ENC = 5; N_ENC_CONV = 3
PRE = 256; DEC = 512; ATT = 128; LOC_F = 32; LOC_K = 31
POST_C = 512; K_POST = 5; N_POST = 5

BH = B // 2
TC = 25
NT = T_DEC // TC
T_PAD = 640


def _zin_body(x_ref, p1_ref, p2_ref, wp_ref, b_ref, o_ref):
    tz = x_ref.shape[0]
    x = x_ref[...].reshape(tz * BH, N_MELS)
    a1 = jnp.maximum(jnp.dot(x, p1_ref[...], preferred_element_type=jnp.float32), 0.0)
    a2 = jnp.maximum(jnp.dot(a1, p2_ref[...], preferred_element_type=jnp.float32), 0.0)
    z = jnp.dot(a2, wp_ref[...], preferred_element_type=jnp.float32) + b_ref[0:1, :]
    o_ref[...] = z.reshape(tz, 1, BH, 4 * DEC)


def _zin_precompute(xsd, p1T, p2T, wpT, cb):
    TZ = 15
    return pl.pallas_call(
        _zin_body,
        grid=(2, T_DEC // TZ),
        in_specs=[
            pl.BlockSpec((TZ, 1, BH, N_MELS), lambda b, t: (t, b, 0, 0)),
            pl.BlockSpec((N_MELS, PRE), lambda b, t: (0, 0)),
            pl.BlockSpec((PRE, PRE), lambda b, t: (0, 0)),
            pl.BlockSpec((PRE, 4 * DEC), lambda b, t: (0, 0)),
            pl.BlockSpec((1, 4 * DEC), lambda b, t: (0, 0)),
        ],
        out_specs=pl.BlockSpec((TZ, 1, BH, 4 * DEC), lambda b, t: (t, b, 0, 0)),
        out_shape=jax.ShapeDtypeStruct((T_DEC, 2, BH, 4 * DEC), jnp.float32),
        compiler_params=pltpu.CompilerParams(
            dimension_semantics=("parallel", "arbitrary")),
        name="zin_precompute",
    )(xsd, p1T, p2T, wpT, cb)


def _pm_body(m_ref, w_ref, o_ref):
    rb = m_ref.shape[1]
    m = m_ref[...].reshape(rb * T_ENC, ENC)
    o_ref[...] = jnp.dot(m, w_ref[...], preferred_element_type=jnp.float32
                         ).reshape(1, rb, T_ENC, ATT)


def _pm_precompute(mem4, amT):
    RB = 8
    return pl.pallas_call(
        _pm_body,
        grid=(2, BH // RB),
        in_specs=[
            pl.BlockSpec((1, RB, T_ENC, ENC), lambda b, r: (b, r, 0, 0)),
            pl.BlockSpec((ENC, ATT), lambda b, r: (0, 0)),
        ],
        out_specs=pl.BlockSpec((1, RB, T_ENC, ATT), lambda b, r: (b, r, 0, 0)),
        out_shape=jax.ShapeDtypeStruct((2, BH, T_ENC, ATT), jnp.float32),
        compiler_params=pltpu.CompilerParams(
            dimension_semantics=("parallel", "arbitrary")),
        name="pm_precompute",
    )(mem4, amT)


def _decoder_body(t_dec, tc,
                  zin_ref, mem_ref, pm_ref, wc_ref, qT_ref, cw_ref, vrep_ref,
                  pg_ref, pgb_ref, maw_ref, mac_ref,
                  ofg_ref, oal_ref,
                  mem_s, pm_s, wc_s, h_s, c_s, ctx_s, ush_s, uac_s, chunk_s,
                  ld_sem, al_sem):
    bb = pl.program_id(0)
    tt = pl.program_id(1)

    @pl.when(tt == 0)
    def _init():
        cp_m = pltpu.make_async_copy(mem_ref.at[bb], mem_s, ld_sem.at[0])
        cp_p = pltpu.make_async_copy(pm_ref.at[bb], pm_s, ld_sem.at[1])
        cp_w = pltpu.make_async_copy(wc_ref, wc_s, ld_sem.at[2])
        cp_m.start(); cp_p.start(); cp_w.start()
        h_s[...] = jnp.zeros_like(h_s)
        c_s[...] = jnp.zeros_like(c_s)
        ctx_s[...] = jnp.zeros_like(ctx_s)
        ush_s[...] = jnp.zeros_like(ush_s)
        uac_s[...] = jnp.zeros_like(uac_s)
        cp_m.wait(); cp_p.wait(); cp_w.wait()

    def step(s, _):
        t_abs = tt * tc + s
        zin = zin_ref[s, 0]
        cx = jnp.concatenate([ctx_s[...], h_s[...]], axis=1)
        z = zin + jnp.dot(cx, wc_s[...], preferred_element_type=jnp.float32)
        i_g = jax.nn.sigmoid(z[:, :DEC])
        f_g = jax.nn.sigmoid(z[:, DEC:2 * DEC])
        g_g = jnp.tanh(z[:, 2 * DEC:3 * DEC])
        o_g = jax.nn.sigmoid(z[:, 3 * DEC:])
        c_new = f_g * c_s[...] + i_g * g_g
        h_new = o_g * jnp.tanh(c_new)
        c_s[...] = c_new
        h_s[...] = h_new

        u = ush_s[...] * maw_ref[...][None] + uac_s[...] * mac_ref[...][None]
        loc = jnp.dot(u.reshape(BH * T_ENC, 128), cw_ref[...],
                      preferred_element_type=jnp.float32)

        q = jnp.dot(h_new, qT_ref[...], preferred_element_type=jnp.float32)
        x = jnp.tanh(pm_s[...] + q[:, None, :] + loc.reshape(BH, T_ENC, ATT))
        e = jnp.dot(x.reshape(BH * T_ENC, ATT), vrep_ref[...],
                    preferred_element_type=jnp.float32).reshape(BH, T_ENC, ATT)

        m = jnp.max(e, axis=1, keepdims=True)
        ex = jnp.exp(e - m)
        al = ex / jnp.sum(ex, axis=1, keepdims=True)

        lane = lax.broadcasted_iota(jnp.int32, (1, 1, 128), 2)
        mm = jnp.where(lane < 31, lane, lane - 31)
        y = pltpu.roll(al, 177, 1)
        for j in range(5):
            y = jnp.where((mm >> j) & 1 == 1, pltpu.roll(y, 1 << j, 1), y)
        ush_s[...] = y
        uac_s[...] = uac_s[...] + y

        ctx_new = jnp.sum(mem_s[...] * al[:, :, 0:1], axis=1)
        ctx_s[...] = ctx_new

        hc = jnp.concatenate([h_new, ctx_new], axis=1)
        frg = jnp.dot(hc, pg_ref[...],
                      preferred_element_type=jnp.float32) + pgb_ref[0:1, :]
        ofg_ref[s, 0] = frg

        lane = lax.broadcasted_iota(jnp.int32, (1, 1, 128), 2)
        sel = lane == (t_abs & 127)
        chunk_s[...] = jnp.where(sel, al, chunk_s[...])

        @pl.when(((t_abs & 127) == 127) | (t_abs == t_dec - 1))
        def _flush():
            base = pl.multiple_of(t_abs - (t_abs & 127), 128)
            cp = pltpu.make_async_copy(
                chunk_s, oal_ref.at[bb, :, :, pl.ds(base, 128)], al_sem)
            cp.start()
            cp.wait()
        return ()

    lax.fori_loop(0, tc, step, ())


def _decoder_call(zin, mem4, pm4, wc, qT, cw, vrep, pg, pgb, maw, mac,
                  t_dec, tc, interpret=False):
    nt = t_dec // tc
    body = functools.partial(_decoder_body, t_dec, tc)
    return pl.pallas_call(
        body,
        grid=(2, nt),
        in_specs=[
            pl.BlockSpec((tc, 1, BH, 4 * DEC), lambda b, t: (t, b, 0, 0)),
            pl.BlockSpec(memory_space=pl.ANY),
            pl.BlockSpec(memory_space=pl.ANY),
            pl.BlockSpec(memory_space=pl.ANY),
            pl.BlockSpec((DEC, ATT), lambda b, t: (0, 0)),
            pl.BlockSpec((128, 128), lambda b, t: (0, 0)),
            pl.BlockSpec((ATT, 128), lambda b, t: (0, 0)),
            pl.BlockSpec((2 * DEC, 128), lambda b, t: (0, 0)),
            pl.BlockSpec((1, 128), lambda b, t: (0, 0)),
            pl.BlockSpec((T_ENC, 128), lambda b, t: (0, 0)),
            pl.BlockSpec((T_ENC, 128), lambda b, t: (0, 0)),
        ],
        out_specs=[
            pl.BlockSpec((tc, 1, BH, 128), lambda b, t: (t, b, 0, 0)),
            pl.BlockSpec(memory_space=pl.ANY),
        ],
        out_shape=[
            jax.ShapeDtypeStruct((t_dec, 2, BH, 128), jnp.float32),
            jax.ShapeDtypeStruct((2, BH, T_ENC, T_PAD), jnp.float32),
        ],
        scratch_shapes=[
            pltpu.VMEM((BH, T_ENC, ENC), jnp.float32),
            pltpu.VMEM((BH, T_ENC, ATT), jnp.float32),
            pltpu.VMEM((2 * DEC, 4 * DEC), jnp.float32),
            pltpu.VMEM((BH, DEC), jnp.float32),
            pltpu.VMEM((BH, DEC), jnp.float32),
            pltpu.VMEM((BH, ENC), jnp.float32),
            pltpu.VMEM((BH, T_ENC, 128), jnp.float32),
            pltpu.VMEM((BH, T_ENC, 128), jnp.float32),
            pltpu.VMEM((BH, T_ENC, 128), jnp.float32),
            pltpu.SemaphoreType.DMA((3,)),
            pltpu.SemaphoreType.DMA,
        ],
        compiler_params=pltpu.CompilerParams(
            dimension_semantics=("parallel", "arbitrary"),
            vmem_limit_bytes=57 * 1024 * 1024,
        ),
        name="tacotron_decoder",
        interpret=interpret,
    )(zin, mem4, pm4, wc, qT, cw, vrep, pg, pgb, maw, mac)


def _conv1d(x, w, b=None, pad=0):
    y = lax.conv_general_dilated(x, w, (1,), [(pad, pad)],
                                 dimension_numbers=('NCH', 'OIH', 'NCH'))
    return y if b is None else y + b[None, :, None]


def _lstm(xs, W):
    H = W['Wh'].shape[1]

    def step(carry, x):
        h, c = carry
        z = x @ W['Wi'].T + h @ W['Wh'].T + W['b']
        i, f, g, o = jnp.split(z, 4, -1)
        c = jax.nn.sigmoid(f) * c + jax.nn.sigmoid(i) * jnp.tanh(g)
        h = jax.nn.sigmoid(o) * jnp.tanh(c)
        return (h, c), h

    Bsz = xs.shape[1]
    init = (jnp.zeros((Bsz, H), xs.dtype), jnp.zeros((Bsz, H), xs.dtype))
    return lax.scan(step, init, xs)[1]


def _loc_weights(p):
    cw_aw = jnp.einsum('fk,af->ka', p['att_loc_conv'][:, 0, :], p['att_loc_lin'])
    cw_ac = jnp.einsum('fk,af->ka', p['att_loc_conv'][:, 1, :], p['att_loc_lin'])
    lanes = jnp.arange(128)
    k_aw = jnp.clip(30 - lanes, 0, 30)
    k_ac = jnp.clip(61 - lanes, 0, 30)
    cw = jnp.where((lanes < 31)[:, None], cw_aw[k_aw], 0.0) + \
         jnp.where(((lanes >= 31) & (lanes < 62))[:, None], cw_ac[k_ac], 0.0)
    t_idx = jnp.arange(T_ENC)[:, None]
    tap_aw = 15 - lanes[None, :]
    tap_ac = 46 - lanes[None, :]
    maw = ((lanes[None, :] < 31) & (t_idx + tap_aw >= 0)
           & (t_idx + tap_aw < T_ENC)).astype(jnp.float32)
    mac = (((lanes[None, :] >= 31) & (lanes[None, :] < 62))
           & (t_idx + tap_ac >= 0) & (t_idx + tap_ac < T_ENC)).astype(jnp.float32)
    return cw, maw, mac


def kernel(texts, encoder_mask, mels_target, params):
    p = params
    x = p['emb'][texts].transpose(0, 2, 1)
    for c in p['enc_convs']:
        x = _conv1d(x, c['w'], c['b'], pad=K_ENC // 2)
        x = jax.nn.relu(x * c['s'][None, :, None] + c['o'][None, :, None])
    xs = x.transpose(2, 0, 1)
    hf = _lstm(xs, p['lstm_f'])
    hb = _lstm(xs[::-1], p['lstm_b'])[::-1]
    memory = jnp.concatenate([hf, hb], -1).transpose(1, 0, 2)
    mem4 = memory.reshape(2, BH, T_ENC, ENC)
    pm4 = _pm_precompute(mem4, p['att_mem'].T)

    dec_in = jnp.concatenate(
        [jnp.zeros((B, 1, N_MELS), mels_target.dtype), mels_target[:, :-1]], 1)
    xsd = dec_in.transpose(1, 0, 2).reshape(T_DEC, 2, BH, N_MELS)
    W = p['cell']
    zin = _zin_precompute(xsd, p['pre1'].T, p['pre2'].T,
                          W['Wi'][:, :PRE].T, W['b'][None, :])
    wc = jnp.concatenate([W['Wi'][:, PRE:].T, W['Wh'].T], axis=0)
    cw, maw, mac = _loc_weights(p)
    vrep = p['att_v'][:, None] * jnp.ones((1, 128), jnp.float32)
    pgw = jnp.concatenate([p['proj_w'], p['gate_w']], axis=0)
    pg = jnp.zeros((2 * DEC, 128), jnp.float32).at[:, :81].set(pgw.T)
    pgb = jnp.zeros((1, 128), jnp.float32).at[0, :80].set(p['proj_b']) \
                                          .at[0, 80].set(p['gate_b'][0])

    ofg, al_pad = _decoder_call(zin, mem4, pm4, wc, p['att_q'].T, cw, vrep,
                                pg, pgb, maw, mac, T_DEC, TC)

    dec_out = ofg[:, :, :, :N_MELS].reshape(T_DEC, B, N_MELS).transpose(1, 0, 2)
    gate_out = ofg[:, :, :, 80].reshape(T_DEC, B).transpose(1, 0)
    alignments = al_pad[:, :, :, :T_DEC].reshape(B, T_ENC, T_DEC) \
                                        .transpose(0, 2, 1)

    y = dec_out.transpose(0, 2, 1)
    for idx, c in enumerate(p['postnet']):
        y = _conv1d(y, c['w'], c['b'], pad=K_POST // 2)
        y = y * c['s'][None, :, None] + c['o'][None, :, None]
        if idx < N_POST - 1:
            y = jnp.tanh(y)
    post_out = y.transpose(0, 2, 1) + dec_out
    return dec_out, post_out, alignments, gate_out


# speedup vs baseline: 3.3390x; 1.0714x over previous
import functools

import jax
import jax.numpy as jnp
from jax import lax
from jax.experimental import pallas as pl
from jax.experimental.pallas import tpu as pltpu

B, T_ENC, T_DEC = 64, 192, 600
VOCAB, N_MELS = 26, 80
EMB = 512; ENC = 512; K_ENC = 5; N_ENC_CONV = 3
PRE = 256; DEC = 512; ATT = 128; LOC_F = 32; LOC_K = 31
POST_C = 512; K_POST = 5; N_POST = 5

BH = B // 2
TC = 25
NT = T_DEC // TC
T_PAD = 640


def _zin_body(x_ref, p1_ref, p2_ref, wp_ref, b_ref, o_ref):
    tz = x_ref.shape[0]
    x = x_ref[...].reshape(tz * BH, N_MELS)
    a1 = jnp.maximum(jnp.dot(x, p1_ref[...], preferred_element_type=jnp.float32), 0.0)
    a2 = jnp.maximum(jnp.dot(a1, p2_ref[...], preferred_element_type=jnp.float32), 0.0)
    z = jnp.dot(a2, wp_ref[...], preferred_element_type=jnp.float32) + b_ref[0:1, :]
    o_ref[...] = z.reshape(tz, 1, BH, 4 * DEC)


def _zin_precompute(xsd, p1T, p2T, wpT, cb):
    TZ = 15
    return pl.pallas_call(
        _zin_body,
        grid=(2, T_DEC // TZ),
        in_specs=[
            pl.BlockSpec((TZ, 1, BH, N_MELS), lambda b, t: (t, b, 0, 0)),
            pl.BlockSpec((N_MELS, PRE), lambda b, t: (0, 0)),
            pl.BlockSpec((PRE, PRE), lambda b, t: (0, 0)),
            pl.BlockSpec((PRE, 4 * DEC), lambda b, t: (0, 0)),
            pl.BlockSpec((1, 4 * DEC), lambda b, t: (0, 0)),
        ],
        out_specs=pl.BlockSpec((TZ, 1, BH, 4 * DEC), lambda b, t: (t, b, 0, 0)),
        out_shape=jax.ShapeDtypeStruct((T_DEC, 2, BH, 4 * DEC), jnp.float32),
        compiler_params=pltpu.CompilerParams(
            dimension_semantics=("parallel", "arbitrary")),
        name="zin_precompute",
    )(xsd, p1T, p2T, wpT, cb)


def _pm_body(m_ref, w_ref, o_ref):
    rb = m_ref.shape[1]
    m = m_ref[...].reshape(rb * T_ENC, ENC)
    o_ref[...] = jnp.dot(m, w_ref[...], preferred_element_type=jnp.float32
                         ).reshape(1, rb, T_ENC, ATT)


def _pm_precompute(mem4, amT):
    RB = 8
    return pl.pallas_call(
        _pm_body,
        grid=(2, BH // RB),
        in_specs=[
            pl.BlockSpec((1, RB, T_ENC, ENC), lambda b, r: (b, r, 0, 0)),
            pl.BlockSpec((ENC, ATT), lambda b, r: (0, 0)),
        ],
        out_specs=pl.BlockSpec((1, RB, T_ENC, ATT), lambda b, r: (b, r, 0, 0)),
        out_shape=jax.ShapeDtypeStruct((2, BH, T_ENC, ATT), jnp.float32),
        compiler_params=pltpu.CompilerParams(
            dimension_semantics=("parallel", "arbitrary")),
        name="pm_precompute",
    )(mem4, amT)


HENC = ENC // 2


def _xin_body(x_ref, w_ref, b_ref, o_ref):
    rb = x_ref.shape[1]
    x = x_ref[...].reshape(rb * B, ENC)
    z = jnp.dot(x, w_ref[0], preferred_element_type=jnp.float32) + b_ref[0]
    o_ref[...] = z.reshape(1, rb, B, 4 * HENC)


def _xin_precompute(xs2, wiT2, b2):
    RB = 24
    return pl.pallas_call(
        _xin_body,
        grid=(2, T_ENC // RB),
        in_specs=[
            pl.BlockSpec((1, RB, B, ENC), lambda d, r: (d, r, 0, 0)),
            pl.BlockSpec((1, ENC, 4 * HENC), lambda d, r: (d, 0, 0)),
            pl.BlockSpec((1, 1, 4 * HENC), lambda d, r: (d, 0, 0)),
        ],
        out_specs=pl.BlockSpec((1, RB, B, 4 * HENC), lambda d, r: (d, r, 0, 0)),
        out_shape=jax.ShapeDtypeStruct((2, T_ENC, B, 4 * HENC), jnp.float32),
        compiler_params=pltpu.CompilerParams(
            dimension_semantics=("parallel", "arbitrary")),
        name="enc_xin",
    )(xs2, wiT2, b2)


def _enc_lstm_body(tce, xin_ref, wh_ref, oh_ref, h_s, c_s):
    tt = pl.program_id(1)

    @pl.when(tt == 0)
    def _init():
        h_s[...] = jnp.zeros_like(h_s)
        c_s[...] = jnp.zeros_like(c_s)

    def step(s, _):
        z = xin_ref[0, s] + jnp.dot(h_s[...], wh_ref[0],
                                    preferred_element_type=jnp.float32)
        i_g = jax.nn.sigmoid(z[:, :HENC])
        f_g = jax.nn.sigmoid(z[:, HENC:2 * HENC])
        g_g = jnp.tanh(z[:, 2 * HENC:3 * HENC])
        o_g = jax.nn.sigmoid(z[:, 3 * HENC:])
        c_new = f_g * c_s[...] + i_g * g_g
        h_new = o_g * jnp.tanh(c_new)
        c_s[...] = c_new
        h_s[...] = h_new
        oh_ref[0, s] = h_new
        return ()

    lax.fori_loop(0, tce, step, ())


def _enc_lstm(xin, whT2):
    TCE = 24
    body = functools.partial(_enc_lstm_body, TCE)
    return pl.pallas_call(
        body,
        grid=(2, T_ENC // TCE),
        in_specs=[
            pl.BlockSpec((1, TCE, B, 4 * HENC), lambda d, t: (d, t, 0, 0)),
            pl.BlockSpec((1, HENC, 4 * HENC), lambda d, t: (d, 0, 0)),
        ],
        out_specs=pl.BlockSpec((1, TCE, B, HENC), lambda d, t: (d, t, 0, 0)),
        out_shape=jax.ShapeDtypeStruct((2, T_ENC, B, HENC), jnp.float32),
        scratch_shapes=[
            pltpu.VMEM((B, HENC), jnp.float32),
            pltpu.VMEM((B, HENC), jnp.float32),
        ],
        compiler_params=pltpu.CompilerParams(
            dimension_semantics=("parallel", "arbitrary")),
        name="enc_lstm",
    )(xin, whT2)


def _decoder_body(t_dec, tc,
                  zin_ref, mem_ref, pm_ref, wc_ref, qT_ref, cw_ref, vrep_ref,
                  pg_ref, pgb_ref, maw_ref, mac_ref,
                  ofg_ref, oal_ref,
                  mem_s, pm_s, wc_s, h_s, c_s, ctx_s, ush_s, uac_s, chunk_s,
                  ld_sem, al_sem):
    bb = pl.program_id(0)
    tt = pl.program_id(1)

    @pl.when(tt == 0)
    def _init():
        cp_m = pltpu.make_async_copy(mem_ref.at[bb], mem_s, ld_sem.at[0])
        cp_p = pltpu.make_async_copy(pm_ref.at[bb], pm_s, ld_sem.at[1])
        cp_w = pltpu.make_async_copy(wc_ref, wc_s, ld_sem.at[2])
        cp_m.start(); cp_p.start(); cp_w.start()
        h_s[...] = jnp.zeros_like(h_s)
        c_s[...] = jnp.zeros_like(c_s)
        ctx_s[...] = jnp.zeros_like(ctx_s)
        ush_s[...] = jnp.zeros_like(ush_s)
        uac_s[...] = jnp.zeros_like(uac_s)
        cp_m.wait(); cp_p.wait(); cp_w.wait()

    def step(s, _):
        t_abs = tt * tc + s
        zin = zin_ref[s, 0]
        cx = jnp.concatenate([ctx_s[...], h_s[...]], axis=1)
        z = zin + jnp.dot(cx, wc_s[...], preferred_element_type=jnp.float32)
        i_g = jax.nn.sigmoid(z[:, :DEC])
        f_g = jax.nn.sigmoid(z[:, DEC:2 * DEC])
        g_g = jnp.tanh(z[:, 2 * DEC:3 * DEC])
        o_g = jax.nn.sigmoid(z[:, 3 * DEC:])
        c_new = f_g * c_s[...] + i_g * g_g
        h_new = o_g * jnp.tanh(c_new)
        c_s[...] = c_new
        h_s[...] = h_new

        u = ush_s[...] * maw_ref[...][None] + uac_s[...] * mac_ref[...][None]
        loc = jnp.dot(u.reshape(BH * T_ENC, 128), cw_ref[...],
                      preferred_element_type=jnp.float32)

        q = jnp.dot(h_new, qT_ref[...], preferred_element_type=jnp.float32)
        x = jnp.tanh(pm_s[...] + q[:, None, :] + loc.reshape(BH, T_ENC, ATT))
        e = jnp.dot(x.reshape(BH * T_ENC, ATT), vrep_ref[...],
                    preferred_element_type=jnp.float32).reshape(BH, T_ENC, ATT)

        m = jnp.max(e, axis=1, keepdims=True)
        ex = jnp.exp(e - m)
        al = ex / jnp.sum(ex, axis=1, keepdims=True)

        lane = lax.broadcasted_iota(jnp.int32, (1, 1, 128), 2)
        mm = jnp.where(lane < 31, lane, lane - 31)
        y = pltpu.roll(al, 177, 1)
        for j in range(5):
            y = jnp.where((mm >> j) & 1 == 1, pltpu.roll(y, 1 << j, 1), y)
        ush_s[...] = y
        uac_s[...] = uac_s[...] + y

        ctx_new = jnp.sum(mem_s[...] * al[:, :, 0:1], axis=1)
        ctx_s[...] = ctx_new

        hc = jnp.concatenate([h_new, ctx_new], axis=1)
        frg = jnp.dot(hc, pg_ref[...],
                      preferred_element_type=jnp.float32) + pgb_ref[0:1, :]
        ofg_ref[s, 0] = frg

        lane = lax.broadcasted_iota(jnp.int32, (1, 1, 128), 2)
        sel = lane == (t_abs & 127)
        chunk_s[...] = jnp.where(sel, al, chunk_s[...])

        @pl.when(((t_abs & 127) == 127) | (t_abs == t_dec - 1))
        def _flush():
            base = pl.multiple_of(t_abs - (t_abs & 127), 128)
            cp = pltpu.make_async_copy(
                chunk_s, oal_ref.at[bb, :, :, pl.ds(base, 128)], al_sem)
            cp.start()
            cp.wait()
        return ()

    lax.fori_loop(0, tc, step, ())


def _decoder_call(zin, mem4, pm4, wc, qT, cw, vrep, pg, pgb, maw, mac,
                  t_dec, tc, interpret=False):
    nt = t_dec // tc
    body = functools.partial(_decoder_body, t_dec, tc)
    return pl.pallas_call(
        body,
        grid=(2, nt),
        in_specs=[
            pl.BlockSpec((tc, 1, BH, 4 * DEC), lambda b, t: (t, b, 0, 0)),
            pl.BlockSpec(memory_space=pl.ANY),
            pl.BlockSpec(memory_space=pl.ANY),
            pl.BlockSpec(memory_space=pl.ANY),
            pl.BlockSpec((DEC, ATT), lambda b, t: (0, 0)),
            pl.BlockSpec((128, 128), lambda b, t: (0, 0)),
            pl.BlockSpec((ATT, 128), lambda b, t: (0, 0)),
            pl.BlockSpec((2 * DEC, 128), lambda b, t: (0, 0)),
            pl.BlockSpec((1, 128), lambda b, t: (0, 0)),
            pl.BlockSpec((T_ENC, 128), lambda b, t: (0, 0)),
            pl.BlockSpec((T_ENC, 128), lambda b, t: (0, 0)),
        ],
        out_specs=[
            pl.BlockSpec((tc, 1, BH, 128), lambda b, t: (t, b, 0, 0)),
            pl.BlockSpec(memory_space=pl.ANY),
        ],
        out_shape=[
            jax.ShapeDtypeStruct((t_dec, 2, BH, 128), jnp.float32),
            jax.ShapeDtypeStruct((2, BH, T_ENC, T_PAD), jnp.float32),
        ],
        scratch_shapes=[
            pltpu.VMEM((BH, T_ENC, ENC), jnp.float32),
            pltpu.VMEM((BH, T_ENC, ATT), jnp.float32),
            pltpu.VMEM((2 * DEC, 4 * DEC), jnp.float32),
            pltpu.VMEM((BH, DEC), jnp.float32),
            pltpu.VMEM((BH, DEC), jnp.float32),
            pltpu.VMEM((BH, ENC), jnp.float32),
            pltpu.VMEM((BH, T_ENC, 128), jnp.float32),
            pltpu.VMEM((BH, T_ENC, 128), jnp.float32),
            pltpu.VMEM((BH, T_ENC, 128), jnp.float32),
            pltpu.SemaphoreType.DMA((3,)),
            pltpu.SemaphoreType.DMA,
        ],
        compiler_params=pltpu.CompilerParams(
            dimension_semantics=("parallel", "arbitrary"),
            vmem_limit_bytes=57 * 1024 * 1024,
        ),
        name="tacotron_decoder",
        interpret=interpret,
    )(zin, mem4, pm4, wc, qT, cw, vrep, pg, pgb, maw, mac)


def _conv1d(x, w, b=None, pad=0):
    y = lax.conv_general_dilated(x, w, (1,), [(pad, pad)],
                                 dimension_numbers=('NCH', 'OIH', 'NCH'))
    return y if b is None else y + b[None, :, None]


def _lstm(xs, W):
    H = W['Wh'].shape[1]

    def step(carry, x):
        h, c = carry
        z = x @ W['Wi'].T + h @ W['Wh'].T + W['b']
        i, f, g, o = jnp.split(z, 4, -1)
        c = jax.nn.sigmoid(f) * c + jax.nn.sigmoid(i) * jnp.tanh(g)
        h = jax.nn.sigmoid(o) * jnp.tanh(c)
        return (h, c), h

    Bsz = xs.shape[1]
    init = (jnp.zeros((Bsz, H), xs.dtype), jnp.zeros((Bsz, H), xs.dtype))
    return lax.scan(step, init, xs)[1]


def _loc_weights(p):
    cw_aw = jnp.einsum('fk,af->ka', p['att_loc_conv'][:, 0, :], p['att_loc_lin'])
    cw_ac = jnp.einsum('fk,af->ka', p['att_loc_conv'][:, 1, :], p['att_loc_lin'])
    lanes = jnp.arange(128)
    k_aw = jnp.clip(30 - lanes, 0, 30)
    k_ac = jnp.clip(61 - lanes, 0, 30)
    cw = jnp.where((lanes < 31)[:, None], cw_aw[k_aw], 0.0) + \
         jnp.where(((lanes >= 31) & (lanes < 62))[:, None], cw_ac[k_ac], 0.0)
    t_idx = jnp.arange(T_ENC)[:, None]
    tap_aw = 15 - lanes[None, :]
    tap_ac = 46 - lanes[None, :]
    maw = ((lanes[None, :] < 31) & (t_idx + tap_aw >= 0)
           & (t_idx + tap_aw < T_ENC)).astype(jnp.float32)
    mac = (((lanes[None, :] >= 31) & (lanes[None, :] < 62))
           & (t_idx + tap_ac >= 0) & (t_idx + tap_ac < T_ENC)).astype(jnp.float32)
    return cw, maw, mac


def kernel(texts, encoder_mask, mels_target, params):
    p = params
    x = p['emb'][texts].transpose(0, 2, 1)
    for c in p['enc_convs']:
        x = _conv1d(x, c['w'], c['b'], pad=K_ENC // 2)
        x = jax.nn.relu(x * c['s'][None, :, None] + c['o'][None, :, None])
    xs = x.transpose(2, 0, 1)
    xs2 = jnp.stack([xs, xs[::-1]], 0)
    wiT2 = jnp.stack([p['lstm_f']['Wi'].T, p['lstm_b']['Wi'].T], 0)
    b2 = jnp.stack([p['lstm_f']['b'][None], p['lstm_b']['b'][None]], 0)
    whT2 = jnp.stack([p['lstm_f']['Wh'].T, p['lstm_b']['Wh'].T], 0)
    xin = jnp.einsum('dtbe,deh->dtbh', xs2, wiT2) + b2[:, None]

    def _scan_dir(xin_d, whT):
        def stp(carry, zi):
            h, c = carry
            z = zi + h @ whT
            i, f, g, o = jnp.split(z, 4, -1)
            c = jax.nn.sigmoid(f) * c + jax.nn.sigmoid(i) * jnp.tanh(g)
            h = jax.nn.sigmoid(o) * jnp.tanh(c)
            return (h, c), h
        init = (jnp.zeros((B, HENC)), jnp.zeros((B, HENC)))
        return lax.scan(stp, init, xin_d)[1]

    oh = jnp.stack([_scan_dir(xin[0], whT2[0]), _scan_dir(xin[1], whT2[1])], 0)
    memory = jnp.concatenate([oh[0], oh[1][::-1]], -1).transpose(1, 0, 2)
    mem4 = memory.reshape(2, BH, T_ENC, ENC)
    pm4 = _pm_precompute(mem4, p['att_mem'].T)

    dec_in = jnp.concatenate(
        [jnp.zeros((B, 1, N_MELS), mels_target.dtype), mels_target[:, :-1]], 1)
    xsd = dec_in.transpose(1, 0, 2).reshape(T_DEC, 2, BH, N_MELS)
    W = p['cell']
    zin = _zin_precompute(xsd, p['pre1'].T, p['pre2'].T,
                          W['Wi'][:, :PRE].T, W['b'][None, :])
    wc = jnp.concatenate([W['Wi'][:, PRE:].T, W['Wh'].T], axis=0)
    cw, maw, mac = _loc_weights(p)
    vrep = p['att_v'][:, None] * jnp.ones((1, 128), jnp.float32)
    pgw = jnp.concatenate([p['proj_w'], p['gate_w']], axis=0)
    pg = jnp.zeros((2 * DEC, 128), jnp.float32).at[:, :81].set(pgw.T)
    pgb = jnp.zeros((1, 128), jnp.float32).at[0, :80].set(p['proj_b']) \
                                          .at[0, 80].set(p['gate_b'][0])

    ofg, al_pad = _decoder_call(zin, mem4, pm4, wc, p['att_q'].T, cw, vrep,
                                pg, pgb, maw, mac, T_DEC, TC)

    dec_out = ofg[:, :, :, :N_MELS].reshape(T_DEC, B, N_MELS).transpose(1, 0, 2)
    gate_out = ofg[:, :, :, 80].reshape(T_DEC, B).transpose(1, 0)
    alignments = al_pad[:, :, :, :T_DEC].reshape(B, T_ENC, T_DEC) \
                                        .transpose(0, 2, 1)

    y = dec_out.transpose(0, 2, 1)
    for idx, c in enumerate(p['postnet']):
        y = _conv1d(y, c['w'], c['b'], pad=K_POST // 2)
        y = y * c['s'][None, :, None] + c['o'][None, :, None]
        if idx < N_POST - 1:
            y = jnp.tanh(y)
    post_out = y.transpose(0, 2, 1) + dec_out
    return dec_out, post_out, alignments, gate_out


# speedup vs baseline: 3.6820x; 1.1027x over previous
import functools

import jax
import jax.numpy as jnp
from jax import lax
from jax.experimental import pallas as pl
from jax.experimental.pallas import tpu as pltpu

B, T_ENC, T_DEC = 64, 192, 600
VOCAB, N_MELS = 26, 80
EMB = 512; ENC = 512; K_ENC = 5; N_ENC_CONV = 3
PRE = 256; DEC = 512; ATT = 128; LOC_F = 32; LOC_K = 31
POST_C = 512; K_POST = 5; N_POST = 5

BH = B // 2
TC = 25
NT = T_DEC // TC
T_PAD = 640


def _zin_body(x_ref, p1_ref, p2_ref, wp_ref, b_ref, o_ref):
    tz = x_ref.shape[0]
    x = x_ref[...].reshape(tz * BH, N_MELS)
    a1 = jnp.maximum(jnp.dot(x, p1_ref[...], preferred_element_type=jnp.float32), 0.0)
    a2 = jnp.maximum(jnp.dot(a1, p2_ref[...], preferred_element_type=jnp.float32), 0.0)
    z = jnp.dot(a2, wp_ref[...], preferred_element_type=jnp.float32) + b_ref[0:1, :]
    o_ref[...] = z.reshape(tz, 1, BH, 4 * DEC)


def _zin_precompute(xsd, p1T, p2T, wpT, cb):
    TZ = 15
    return pl.pallas_call(
        _zin_body,
        grid=(2, T_DEC // TZ),
        in_specs=[
            pl.BlockSpec((TZ, 1, BH, N_MELS), lambda b, t: (t, b, 0, 0)),
            pl.BlockSpec((N_MELS, PRE), lambda b, t: (0, 0)),
            pl.BlockSpec((PRE, PRE), lambda b, t: (0, 0)),
            pl.BlockSpec((PRE, 4 * DEC), lambda b, t: (0, 0)),
            pl.BlockSpec((1, 4 * DEC), lambda b, t: (0, 0)),
        ],
        out_specs=pl.BlockSpec((TZ, 1, BH, 4 * DEC), lambda b, t: (t, b, 0, 0)),
        out_shape=jax.ShapeDtypeStruct((T_DEC, 2, BH, 4 * DEC), jnp.float32),
        compiler_params=pltpu.CompilerParams(
            dimension_semantics=("parallel", "arbitrary")),
        name="zin_precompute",
    )(xsd, p1T, p2T, wpT, cb)


def _pm_body(m_ref, w_ref, o_ref):
    rb = m_ref.shape[1]
    m = m_ref[...].reshape(rb * T_ENC, ENC)
    o_ref[...] = jnp.dot(m, w_ref[...], preferred_element_type=jnp.float32
                         ).reshape(1, rb, T_ENC, ATT)


def _pm_precompute(mem4, amT):
    RB = 8
    return pl.pallas_call(
        _pm_body,
        grid=(2, BH // RB),
        in_specs=[
            pl.BlockSpec((1, RB, T_ENC, ENC), lambda b, r: (b, r, 0, 0)),
            pl.BlockSpec((ENC, ATT), lambda b, r: (0, 0)),
        ],
        out_specs=pl.BlockSpec((1, RB, T_ENC, ATT), lambda b, r: (b, r, 0, 0)),
        out_shape=jax.ShapeDtypeStruct((2, BH, T_ENC, ATT), jnp.float32),
        compiler_params=pltpu.CompilerParams(
            dimension_semantics=("parallel", "arbitrary")),
        name="pm_precompute",
    )(mem4, amT)


HENC = ENC // 2


def _xin_body(x_ref, w_ref, b_ref, o_ref):
    rb = x_ref.shape[1]
    x = x_ref[...].reshape(rb * B, ENC)
    z = jnp.dot(x, w_ref[0], preferred_element_type=jnp.float32) + b_ref[0]
    o_ref[...] = z.reshape(1, rb, B, 4 * HENC)


def _xin_precompute(xs2, wiT2, b2):
    RB = 24
    return pl.pallas_call(
        _xin_body,
        grid=(2, T_ENC // RB),
        in_specs=[
            pl.BlockSpec((1, RB, B, ENC), lambda d, r: (d, r, 0, 0)),
            pl.BlockSpec((1, ENC, 4 * HENC), lambda d, r: (d, 0, 0)),
            pl.BlockSpec((1, 1, 4 * HENC), lambda d, r: (d, 0, 0)),
        ],
        out_specs=pl.BlockSpec((1, RB, B, 4 * HENC), lambda d, r: (d, r, 0, 0)),
        out_shape=jax.ShapeDtypeStruct((2, T_ENC, B, 4 * HENC), jnp.float32),
        compiler_params=pltpu.CompilerParams(
            dimension_semantics=("parallel", "arbitrary")),
        name="enc_xin",
    )(xs2, wiT2, b2)


def _enc_lstm_body(tce, xin_ref, wh_ref, oh_ref, h_s, c_s):
    tt = pl.program_id(1)

    @pl.when(tt == 0)
    def _init():
        h_s[...] = jnp.zeros_like(h_s)
        c_s[...] = jnp.zeros_like(c_s)

    def step(s, _):
        z = xin_ref[0, s] + jnp.dot(h_s[...], wh_ref[0],
                                    preferred_element_type=jnp.float32)
        i_g = jax.nn.sigmoid(z[:, :HENC])
        f_g = jax.nn.sigmoid(z[:, HENC:2 * HENC])
        g_g = jnp.tanh(z[:, 2 * HENC:3 * HENC])
        o_g = jax.nn.sigmoid(z[:, 3 * HENC:])
        c_new = f_g * c_s[...] + i_g * g_g
        h_new = o_g * jnp.tanh(c_new)
        c_s[...] = c_new
        h_s[...] = h_new
        oh_ref[0, s] = h_new
        return ()

    lax.fori_loop(0, tce, step, ())


def _enc_lstm(xin, whT2):
    TCE = 24
    body = functools.partial(_enc_lstm_body, TCE)
    return pl.pallas_call(
        body,
        grid=(2, T_ENC // TCE),
        in_specs=[
            pl.BlockSpec((1, TCE, B, 4 * HENC), lambda d, t: (d, t, 0, 0)),
            pl.BlockSpec((1, HENC, 4 * HENC), lambda d, t: (d, 0, 0)),
        ],
        out_specs=pl.BlockSpec((1, TCE, B, HENC), lambda d, t: (d, t, 0, 0)),
        out_shape=jax.ShapeDtypeStruct((2, T_ENC, B, HENC), jnp.float32),
        scratch_shapes=[
            pltpu.VMEM((B, HENC), jnp.float32),
            pltpu.VMEM((B, HENC), jnp.float32),
        ],
        compiler_params=pltpu.CompilerParams(
            dimension_semantics=("parallel", "arbitrary")),
        name="enc_lstm",
    )(xin, whT2)


def _decoder_body(t_dec, tc,
                  zin_ref, mem_ref, pm_ref, wc_ref, qT_ref, cw_ref, vrep_ref,
                  pg_ref, pgb_ref, maw_ref, mac_ref,
                  ofg_ref, oal_ref,
                  mem_s, pm_s, wc_s, h_s, c_s, ctx_s, ush_s, uac_s, chunk_s,
                  ld_sem, al_sem):
    bb = pl.program_id(0)
    tt = pl.program_id(1)

    @pl.when(tt == 0)
    def _init():
        cp_m = pltpu.make_async_copy(mem_ref.at[bb], mem_s, ld_sem.at[0])
        cp_p = pltpu.make_async_copy(pm_ref.at[bb], pm_s, ld_sem.at[1])
        cp_w = pltpu.make_async_copy(wc_ref, wc_s, ld_sem.at[2])
        cp_m.start(); cp_p.start(); cp_w.start()
        h_s[...] = jnp.zeros_like(h_s)
        c_s[...] = jnp.zeros_like(c_s)
        ctx_s[...] = jnp.zeros_like(ctx_s)
        ush_s[...] = jnp.zeros_like(ush_s)
        uac_s[...] = jnp.zeros_like(uac_s)
        cp_m.wait(); cp_p.wait(); cp_w.wait()

    def step(s, _):
        t_abs = tt * tc + s
        zin = zin_ref[s, 0]
        cx = jnp.concatenate([ctx_s[...], h_s[...]], axis=1)
        z = zin + jnp.dot(cx, wc_s[...], preferred_element_type=jnp.float32)
        i_g = jax.nn.sigmoid(z[:, :DEC])
        f_g = jax.nn.sigmoid(z[:, DEC:2 * DEC])
        g_g = jnp.tanh(z[:, 2 * DEC:3 * DEC])
        o_g = jax.nn.sigmoid(z[:, 3 * DEC:])
        c_new = f_g * c_s[...] + i_g * g_g
        h_new = o_g * jnp.tanh(c_new)
        c_s[...] = c_new
        h_s[...] = h_new

        u = ush_s[...] * maw_ref[...][None] + uac_s[...] * mac_ref[...][None]
        loc = jnp.dot(u.reshape(BH * T_ENC, 128), cw_ref[...],
                      preferred_element_type=jnp.float32)

        q = jnp.dot(h_new, qT_ref[...], preferred_element_type=jnp.float32)
        x = jnp.tanh(pm_s[...] + q[:, None, :] + loc.reshape(BH, T_ENC, ATT))
        e = jnp.dot(x.reshape(BH * T_ENC, ATT), vrep_ref[...],
                    preferred_element_type=jnp.float32).reshape(BH, T_ENC, ATT)

        m = jnp.max(e, axis=1, keepdims=True)
        ex = jnp.exp(e - m)
        al = ex / jnp.sum(ex, axis=1, keepdims=True)

        lane = lax.broadcasted_iota(jnp.int32, (1, 1, 128), 2)
        mm = jnp.where(lane < 31, lane, lane - 31)
        y = pltpu.roll(al, 177, 1)
        for j in range(5):
            y = jnp.where((mm >> j) & 1 == 1, pltpu.roll(y, 1 << j, 1), y)
        ush_s[...] = y
        uac_s[...] = uac_s[...] + y

        ctx_new = jnp.sum(mem_s[...] * al[:, :, 0:1], axis=1)
        ctx_s[...] = ctx_new

        hc = jnp.concatenate([h_new, ctx_new], axis=1)
        frg = jnp.dot(hc, pg_ref[...],
                      preferred_element_type=jnp.float32) + pgb_ref[0:1, :]
        ofg_ref[s, 0] = frg

        lane = lax.broadcasted_iota(jnp.int32, (1, 1, 128), 2)
        sel = lane == (t_abs & 127)
        chunk_s[...] = jnp.where(sel, al, chunk_s[...])

        @pl.when(((t_abs & 127) == 127) | (t_abs == t_dec - 1))
        def _flush():
            base = pl.multiple_of(t_abs - (t_abs & 127), 128)
            cp = pltpu.make_async_copy(
                chunk_s, oal_ref.at[bb, :, :, pl.ds(base, 128)], al_sem)
            cp.start()
            cp.wait()
        return ()

    lax.fori_loop(0, tc, step, ())


def _decoder_call(zin, mem4, pm4, wc, qT, cw, vrep, pg, pgb, maw, mac,
                  t_dec, tc, interpret=False):
    nt = t_dec // tc
    body = functools.partial(_decoder_body, t_dec, tc)
    return pl.pallas_call(
        body,
        grid=(2, nt),
        in_specs=[
            pl.BlockSpec((tc, 1, BH, 4 * DEC), lambda b, t: (t, b, 0, 0)),
            pl.BlockSpec(memory_space=pl.ANY),
            pl.BlockSpec(memory_space=pl.ANY),
            pl.BlockSpec(memory_space=pl.ANY),
            pl.BlockSpec((DEC, ATT), lambda b, t: (0, 0)),
            pl.BlockSpec((128, 128), lambda b, t: (0, 0)),
            pl.BlockSpec((ATT, 128), lambda b, t: (0, 0)),
            pl.BlockSpec((2 * DEC, 128), lambda b, t: (0, 0)),
            pl.BlockSpec((1, 128), lambda b, t: (0, 0)),
            pl.BlockSpec((T_ENC, 128), lambda b, t: (0, 0)),
            pl.BlockSpec((T_ENC, 128), lambda b, t: (0, 0)),
        ],
        out_specs=[
            pl.BlockSpec((tc, 1, BH, 128), lambda b, t: (t, b, 0, 0)),
            pl.BlockSpec(memory_space=pl.ANY),
        ],
        out_shape=[
            jax.ShapeDtypeStruct((t_dec, 2, BH, 128), jnp.float32),
            jax.ShapeDtypeStruct((2, BH, T_ENC, T_PAD), jnp.float32),
        ],
        scratch_shapes=[
            pltpu.VMEM((BH, T_ENC, ENC), jnp.float32),
            pltpu.VMEM((BH, T_ENC, ATT), jnp.float32),
            pltpu.VMEM((2 * DEC, 4 * DEC), jnp.float32),
            pltpu.VMEM((BH, DEC), jnp.float32),
            pltpu.VMEM((BH, DEC), jnp.float32),
            pltpu.VMEM((BH, ENC), jnp.float32),
            pltpu.VMEM((BH, T_ENC, 128), jnp.float32),
            pltpu.VMEM((BH, T_ENC, 128), jnp.float32),
            pltpu.VMEM((BH, T_ENC, 128), jnp.float32),
            pltpu.SemaphoreType.DMA((3,)),
            pltpu.SemaphoreType.DMA,
        ],
        compiler_params=pltpu.CompilerParams(
            dimension_semantics=("parallel", "arbitrary"),
            vmem_limit_bytes=57 * 1024 * 1024,
        ),
        name="tacotron_decoder",
        interpret=interpret,
    )(zin, mem4, pm4, wc, qT, cw, vrep, pg, pgb, maw, mac)


def _conv1d(x, w, b=None, pad=0):
    y = lax.conv_general_dilated(x, w, (1,), [(pad, pad)],
                                 dimension_numbers=('NCH', 'OIH', 'NCH'))
    return y if b is None else y + b[None, :, None]


def _lstm(xs, W):
    H = W['Wh'].shape[1]

    def step(carry, x):
        h, c = carry
        z = x @ W['Wi'].T + h @ W['Wh'].T + W['b']
        i, f, g, o = jnp.split(z, 4, -1)
        c = jax.nn.sigmoid(f) * c + jax.nn.sigmoid(i) * jnp.tanh(g)
        h = jax.nn.sigmoid(o) * jnp.tanh(c)
        return (h, c), h

    Bsz = xs.shape[1]
    init = (jnp.zeros((Bsz, H), xs.dtype), jnp.zeros((Bsz, H), xs.dtype))
    return lax.scan(step, init, xs)[1]


def _loc_weights(p):
    cw_aw = jnp.einsum('fk,af->ka', p['att_loc_conv'][:, 0, :], p['att_loc_lin'])
    cw_ac = jnp.einsum('fk,af->ka', p['att_loc_conv'][:, 1, :], p['att_loc_lin'])
    lanes = jnp.arange(128)
    k_aw = jnp.clip(30 - lanes, 0, 30)
    k_ac = jnp.clip(61 - lanes, 0, 30)
    cw = jnp.where((lanes < 31)[:, None], cw_aw[k_aw], 0.0) + \
         jnp.where(((lanes >= 31) & (lanes < 62))[:, None], cw_ac[k_ac], 0.0)
    t_idx = jnp.arange(T_ENC)[:, None]
    tap_aw = 15 - lanes[None, :]
    tap_ac = 46 - lanes[None, :]
    maw = ((lanes[None, :] < 31) & (t_idx + tap_aw >= 0)
           & (t_idx + tap_aw < T_ENC)).astype(jnp.float32)
    mac = (((lanes[None, :] >= 31) & (lanes[None, :] < 62))
           & (t_idx + tap_ac >= 0) & (t_idx + tap_ac < T_ENC)).astype(jnp.float32)
    return cw, maw, mac


def kernel(texts, encoder_mask, mels_target, params):
    p = params
    x = p['emb'][texts].transpose(0, 2, 1)
    for c in p['enc_convs']:
        x = _conv1d(x, c['w'], c['b'], pad=K_ENC // 2)
        x = jax.nn.relu(x * c['s'][None, :, None] + c['o'][None, :, None])
    xs = x.transpose(2, 0, 1)
    xs2 = jnp.stack([xs, xs[::-1]], 0)
    wiT2 = jnp.stack([p['lstm_f']['Wi'].T, p['lstm_b']['Wi'].T], 0)
    b2 = jnp.stack([p['lstm_f']['b'][None], p['lstm_b']['b'][None]], 0)
    whT2 = jnp.stack([p['lstm_f']['Wh'].T, p['lstm_b']['Wh'].T], 0)
    xin = jnp.einsum('dtbe,deh->dtbh', xs2, wiT2) + b2[:, None]
    oh = _enc_lstm(xin, whT2)
    memory = jnp.concatenate([oh[0], oh[1][::-1]], -1).transpose(1, 0, 2)
    mem4 = memory.reshape(2, BH, T_ENC, ENC)
    pm4 = _pm_precompute(mem4, p['att_mem'].T)

    dec_in = jnp.concatenate(
        [jnp.zeros((B, 1, N_MELS), mels_target.dtype), mels_target[:, :-1]], 1)
    xsd = dec_in.transpose(1, 0, 2).reshape(T_DEC, 2, BH, N_MELS)
    W = p['cell']
    zin = _zin_precompute(xsd, p['pre1'].T, p['pre2'].T,
                          W['Wi'][:, :PRE].T, W['b'][None, :])
    wc = jnp.concatenate([W['Wi'][:, PRE:].T, W['Wh'].T], axis=0)
    cw, maw, mac = _loc_weights(p)
    vrep = p['att_v'][:, None] * jnp.ones((1, 128), jnp.float32)
    pgw = jnp.concatenate([p['proj_w'], p['gate_w']], axis=0)
    pg = jnp.zeros((2 * DEC, 128), jnp.float32).at[:, :81].set(pgw.T)
    pgb = jnp.zeros((1, 128), jnp.float32).at[0, :80].set(p['proj_b']) \
                                          .at[0, 80].set(p['gate_b'][0])

    ofg, al_pad = _decoder_call(zin, mem4, pm4, wc, p['att_q'].T, cw, vrep,
                                pg, pgb, maw, mac, T_DEC, TC)

    dec_out = ofg[:, :, :, :N_MELS].reshape(T_DEC, B, N_MELS).transpose(1, 0, 2)
    gate_out = ofg[:, :, :, 80].reshape(T_DEC, B).transpose(1, 0)
    alignments = al_pad[:, :, :, :T_DEC].reshape(B, T_ENC, T_DEC) \
                                        .transpose(0, 2, 1)

    y = dec_out.transpose(0, 2, 1)
    for idx, c in enumerate(p['postnet']):
        y = _conv1d(y, c['w'], c['b'], pad=K_POST // 2)
        y = y * c['s'][None, :, None] + c['o'][None, :, None]
        if idx < N_POST - 1:
            y = jnp.tanh(y)
    post_out = y.transpose(0, 2, 1) + dec_out
    return dec_out, post_out, alignments, gate_out
